# baseline (speedup 1.0000x reference)
import functools

import jax
import jax.numpy as jnp
from jax import lax
from jax.experimental import pallas as pl
from jax.experimental.pallas import tpu as pltpu
from jax.experimental.pallas import tpu_sc as plsc

NW = 32
BB = 128
NROW = 4
NTR = 2
DP = 128


def _gather_kernel(L, d, table_hbm, idx_hbm, out_hbm,
                   idx_v, rows_v, tr_v, gsem, wsem):
    wid = lax.axis_index("s") * 2 + lax.axis_index("c")
    b0 = wid * BB
    pltpu.sync_copy(idx_hbm.at[wid], idx_v)
    lanes = lax.iota(jnp.int32, 16)

    def gather_copy(l, p):
        return pltpu.make_async_copy(
            table_hbm.at[idx_v.at[l]], rows_v.at[p], gsem)

    def write_copy(l, q):
        dst = out_hbm.at[l, :, pl.ds(b0, BB)]
        return pltpu.make_async_copy(tr_v.at[q], dst, wsem)

    gather_copy(0, 0).start()
    gather_copy(1, 1).start()

    @pl.loop(0, L, step=NROW)
    def _(l0):
        for dp in range(NROW):
            l = l0 + dp
            p = dp
            q = dp % NTR
            gather_copy(l, p).wait()

            @pl.when(l >= NTR)
            def _():
                write_copy(l - NTR, q).wait()

            @pl.loop(0, d)
            def _(dd):
                dsplat = jnp.full((16,), dd, jnp.int32)
                for k in range(BB // 16):
                    vec = plsc.load_gather(
                        rows_v, [jnp.full((16,), p, jnp.int32),
                                 k * 16 + lanes, dsplat])
                    tr_v[q, dd, pl.ds(k * 16, 16)] = vec

            write_copy(l, q).start()

            @pl.when(l + 2 < L)
            def _():
                gather_copy(l + 2, (dp + 2) % NROW).start()

    write_copy(L - 2, 0).wait()
    write_copy(L - 1, 1).wait()


def kernel(characters, tokens, table):
    B, L = tokens.shape
    V, D = table.shape

    tab128 = jnp.pad(table, ((0, 0), (0, DP - D)))
    idx = tokens.T.reshape(L, NW, BB).transpose(1, 0, 2).astype(jnp.int32)

    mesh = plsc.VectorSubcoreMesh(core_axis_name="c", subcore_axis_name="s")
    run = functools.partial(
        pl.kernel,
        out_type=jax.ShapeDtypeStruct((L, D, B), jnp.float32),
        mesh=mesh,
        compiler_params=pltpu.CompilerParams(
            use_tc_tiling_on_sc=True, needs_layout_passes=False),
        scratch_types=[
            pltpu.VMEM((L, BB), jnp.int32),
            pltpu.VMEM((NROW, BB, DP), jnp.float32),
            pltpu.VMEM((NTR, D, BB), jnp.float32),
            pltpu.SemaphoreType.DMA,
            pltpu.SemaphoreType.DMA,
        ],
    )(functools.partial(_gather_kernel, L, D))

    out = run(tab128, idx)
    return out.transpose(2, 0, 1)


# speedup vs baseline: 1.3919x; 1.3919x over previous
import functools

import jax
import jax.numpy as jnp
from jax import lax
from jax.experimental import pallas as pl
from jax.experimental.pallas import tpu as pltpu
from jax.experimental.pallas import tpu_sc as plsc

NW = 32
BB = 128
NROW = 3
NTR = 2
LP = 128
RP = 129


def _gather_kernel(L, d, table_hbm, div_hbm, par_hbm, out_hbm,
                   div_v, par_v, rows_v, tr_v, gsem, wsem):
    wid = lax.axis_index("s") * 2 + lax.axis_index("c")
    b0 = wid * BB
    pltpu.sync_copy(div_hbm.at[wid], div_v)
    pltpu.sync_copy(par_hbm.at[wid], par_v)
    lanes = lax.iota(jnp.int32, 16)

    def gather_copy(l, p):
        return pltpu.make_async_copy(
            table_hbm.at[div_v.at[l]], rows_v.at[p], gsem)

    def write_copy(l, q):
        src = tr_v.at[q, :, pl.ds(0, BB)]
        dst = out_hbm.at[l, :, pl.ds(b0, BB)]
        return pltpu.make_async_copy(src, dst, wsem)

    gather_copy(0, 0).start()
    gather_copy(1, 1).start()

    @pl.loop(0, L)
    def _(l):
        p = lax.rem(l, NROW)
        q = lax.rem(l, NTR)
        gather_copy(l, p).wait()

        @pl.when(l >= NTR)
        def _():
            write_copy(l - NTR, q).wait()

        qs = jnp.full((16,), q, jnp.int32)

        @pl.loop(0, BB // 16)
        def _(bg):
            parvec = par_v[l, pl.ds(bg * 16, 16)]
            for k in range(16):
                b = bg * 16 + k
                pk = parvec[k]
                bs = jnp.full((16,), b, jnp.int32)
                for j in range(d // 16):
                    vec = rows_v[p, b, pl.ds(pk + j * 16, 16)]
                    plsc.store_scatter(
                        tr_v, [qs, j * 16 + lanes, bs], vec)

        write_copy(l, q).start()

        @pl.when(l + 2 < L)
        def _():
            gather_copy(l + 2, lax.rem(l + 2, NROW)).start()

    write_copy(L - 2, lax.rem(L - 2, NTR)).wait()
    write_copy(L - 1, lax.rem(L - 1, NTR)).wait()


def kernel(characters, tokens, table):
    B, L = tokens.shape
    V, D = table.shape

    tab2 = table.reshape(V // 2, 2 * D)
    idx = tokens.T.reshape(L, NW, BB).transpose(1, 0, 2).astype(jnp.int32)
    div = idx // 2
    par = (idx % 2) * D

    mesh = plsc.VectorSubcoreMesh(core_axis_name="c", subcore_axis_name="s")
    run = functools.partial(
        pl.kernel,
        out_type=jax.ShapeDtypeStruct((L, D, B), jnp.float32),
        mesh=mesh,
        compiler_params=pltpu.CompilerParams(
            use_tc_tiling_on_sc=False, needs_layout_passes=False),
        scratch_types=[
            pltpu.VMEM((L, BB), jnp.int32),
            pltpu.VMEM((L, BB), jnp.int32),
            pltpu.VMEM((NROW, BB, LP), jnp.float32),
            pltpu.VMEM((NTR, D, RP), jnp.float32),
            pltpu.SemaphoreType.DMA,
            pltpu.SemaphoreType.DMA,
        ],
    )(functools.partial(_gather_kernel, L, D))

    out = run(tab2, div, par)
    return out.transpose(2, 0, 1)
